# baseline (speedup 1.0000x reference)
import functools

import jax
import jax.numpy as jnp
from jax import lax
from jax.experimental import pallas as pl
from jax.experimental.pallas import tpu as pltpu

HIDDEN = 1024
NUM_EXPERTS = 8
EXPERT_SIZE = 2048
TOKENS = 2048
BLOCK = 256
NBLOCKS = TOKENS // BLOCK + NUM_EXPERTS - 1
PADDED = NBLOCKS * BLOCK


def _gate_body(x_ref, wg_ref, bg_ref, o_ref):
    logits = lax.dot_general(
        x_ref[...], wg_ref[...], (((1,), (1,)), ((), ())),
        preferred_element_type=jnp.float32)
    logits = logits + bg_ref[...]
    o_ref[...] = jnp.argmax(logits, axis=1).astype(jnp.int32)[:, None]


def _gate(x_flat, Wg, bg):
    eidx = pl.pallas_call(
        _gate_body,
        out_shape=jax.ShapeDtypeStruct((TOKENS, 1), jnp.int32),
    )(x_flat, Wg, bg.reshape(1, NUM_EXPERTS))
    return eidx.reshape(TOKENS)


def _mlp_body(be_ref, x_ref, w1_ref, b1_ref, w2_ref, b2_ref, o_ref):
    h = lax.dot_general(
        x_ref[...], w1_ref[0], (((1,), (1,)), ((), ())),
        preferred_element_type=jnp.float32)
    h = jnp.maximum(h + b1_ref[0], 0.0)
    y = lax.dot_general(
        h, w2_ref[0], (((1,), (1,)), ((), ())),
        preferred_element_type=jnp.float32)
    o_ref[...] = y + b2_ref[0]


def _expert_mlp(x_sorted, W1, b1, W2, b2, block_expert):
    grid_spec = pltpu.PrefetchScalarGridSpec(
        num_scalar_prefetch=1,
        grid=(NBLOCKS,),
        in_specs=[
            pl.BlockSpec((BLOCK, HIDDEN), lambda i, be: (i, 0)),
            pl.BlockSpec((1, EXPERT_SIZE, HIDDEN), lambda i, be: (be[i], 0, 0)),
            pl.BlockSpec((1, 1, EXPERT_SIZE), lambda i, be: (be[i], 0, 0)),
            pl.BlockSpec((1, HIDDEN, EXPERT_SIZE), lambda i, be: (be[i], 0, 0)),
            pl.BlockSpec((1, 1, HIDDEN), lambda i, be: (be[i], 0, 0)),
        ],
        out_specs=pl.BlockSpec((BLOCK, HIDDEN), lambda i, be: (i, 0)),
    )
    return pl.pallas_call(
        _mlp_body,
        grid_spec=grid_spec,
        out_shape=jax.ShapeDtypeStruct((PADDED, HIDDEN), jnp.float32),
    )(block_expert, x_sorted, W1,
      b1.reshape(NUM_EXPERTS, 1, EXPERT_SIZE),
      W2, b2.reshape(NUM_EXPERTS, 1, HIDDEN))


def _route(eidx):
    counts = jnp.bincount(eidx, length=NUM_EXPERTS)
    blocks = (counts + BLOCK - 1) // BLOCK
    cumblk = jnp.cumsum(blocks)
    pstart = (cumblk - blocks) * BLOCK
    cstart = jnp.cumsum(counts) - counts
    order = jnp.argsort(eidx, stable=True)
    ranks_sorted = jnp.arange(TOKENS) - cstart[eidx[order]]
    pos_sorted = pstart[eidx[order]] + ranks_sorted
    sorted_pos = jnp.zeros((TOKENS,), jnp.int32).at[order].set(pos_sorted.astype(jnp.int32))
    perm = jnp.zeros((PADDED,), jnp.int32).at[pos_sorted].set(order.astype(jnp.int32))
    block_expert = jnp.minimum(
        jnp.searchsorted(cumblk, jnp.arange(NBLOCKS), side="right"),
        NUM_EXPERTS - 1).astype(jnp.int32)
    return sorted_pos, perm, block_expert


def kernel(x, Wg, bg, W1, b1, W2, b2):
    batch, seq, hidden = x.shape
    x_flat = x.reshape(-1, hidden)
    eidx = _gate(x_flat, Wg, bg)
    sorted_pos, perm, block_expert = _route(eidx)
    x_sorted = x_flat[perm]
    y_sorted = _expert_mlp(x_sorted, W1, b1, W2, b2, block_expert)
    out = y_sorted[sorted_pos]
    return out.reshape(batch, seq, hidden)


# speedup vs baseline: 2.4634x; 2.4634x over previous
import jax
import jax.numpy as jnp
from jax import lax
from jax.experimental import pallas as pl
from jax.experimental.pallas import tpu as pltpu
from jax.experimental.pallas import tpu_sc as plsc

HIDDEN = 1024
NUM_EXPERTS = 8
EXPERT_SIZE = 2048
TOKENS = 2048
BLOCK = 512
NBLOCKS = TOKENS // BLOCK + NUM_EXPERTS - 1
PADDED = NBLOCKS * BLOCK
BE_LEN = 32

NC = 2
NS = 16
NW = NC * NS
LANES = 16
TOK_PER_W = TOKENS // NW

_SC_MESH = plsc.VectorSubcoreMesh(core_axis_name="c", subcore_axis_name="s")


def _gate_body(x_ref, wg_ref, bg_ref, o_ref):
    logits = lax.dot_general(
        x_ref[...], wg_ref[...], (((1,), (1,)), ((), ())),
        preferred_element_type=jnp.float32)
    logits = logits + bg_ref[...]
    o_ref[...] = jnp.argmax(logits, axis=1).astype(jnp.int32)[:, None]


def _gate(x_flat, Wg, bg):
    eidx = pl.pallas_call(
        _gate_body,
        out_shape=jax.ShapeDtypeStruct((TOKENS, 1), jnp.int32),
    )(x_flat, Wg, bg.reshape(1, NUM_EXPERTS))
    return eidx.reshape(TOKENS)


def _route_body(eidx_hbm, x_hbm, pos_hbm, be_hbm, xs_hbm,
                eidx_v, rank_v, pos_v, pos64_v, base_v, be_v, rows_v, sem):
    wid = lax.axis_index("s") * NC + lax.axis_index("c")
    lane = lax.iota(jnp.int32, LANES)

    with jax.named_scope("eidx_copy"):
        pltpu.sync_copy(eidx_hbm, eidx_v)

    tbase = wid * TOK_PER_W
    xread = pltpu.async_copy(x_hbm.at[pl.ds(tbase, TOK_PER_W)], rows_v, sem)

    base_v[...] = jnp.zeros((LANES,), jnp.int32)

    def count_step(g, carry):
        v = eidx_v[pl.ds(g * LANES, LANES)]
        cnt, last = plsc.scan_count(v)
        base = plsc.load_gather(base_v, [v])
        rank_v[pl.ds(g * LANES, LANES)] = base + cnt - 1
        plsc.store_scatter(base_v, [v], base + cnt, mask=last)
        return carry

    with jax.named_scope("pass1"):
        lax.fori_loop(0, TOKENS // LANES, count_step, 0)

    counts = base_v[...]
    blocks = (counts + (BLOCK - 1)) // BLOCK
    cumblk = plsc.cumsum(blocks)
    base_v[...] = (cumblk - blocks) * BLOCK

    be_v[pl.ds(0, LANES)] = jnp.zeros((LANES,), jnp.int32)
    be_v[pl.ds(LANES, LANES)] = jnp.zeros((LANES,), jnp.int32)

    def pos_step(g, carry):
        v = eidx_v[pl.ds(g * LANES, LANES)]
        r = rank_v[pl.ds(g * LANES, LANES)]
        p = r + plsc.load_gather(base_v, [v])
        pos_v[pl.ds(g * LANES, LANES)] = p
        plsc.store_scatter(be_v, [p // BLOCK], v)
        return carry

    with jax.named_scope("pass2"):
        lax.fori_loop(0, TOKENS // LANES, pos_step, 0)

    pltpu.sync_copy(pos_v.at[pl.ds(tbase, TOK_PER_W)],
                    pos_hbm.at[pl.ds(tbase, TOK_PER_W)])

    @pl.when(wid == 0)
    def _():
        nused = jnp.full((LANES,), jnp.max(cumblk))
        last_e = plsc.load_gather(be_v, [nused - 1])
        for gi in range(BE_LEN // LANES):
            bidx = lane + gi * LANES
            cur = be_v[pl.ds(gi * LANES, LANES)]
            keep = ((bidx - nused) >> 31) & 1
            fixed = keep * cur + (1 - keep) * last_e
            fixed = jnp.where(bidx == NBLOCKS, nused, fixed)
            be_v[pl.ds(gi * LANES, LANES)] = fixed
        pltpu.sync_copy(be_v, be_hbm)

    with jax.named_scope("rowscatter"):
        for i in range(TOK_PER_W // LANES):
            pos64_v[pl.ds(i * LANES, LANES)] = (
                pos_v[pl.ds(tbase + i * LANES, LANES)])
        xread.wait()
        pltpu.async_copy(rows_v, xs_hbm.at[pos64_v], sem).wait()


def _route_and_gather(eidx, x_flat):
    return pl.kernel(
        _route_body,
        out_type=[
            jax.ShapeDtypeStruct((TOKENS,), jnp.int32),
            jax.ShapeDtypeStruct((BE_LEN,), jnp.int32),
            jax.ShapeDtypeStruct((PADDED, HIDDEN), jnp.float32),
        ],
        mesh=_SC_MESH,
        compiler_params=pltpu.CompilerParams(needs_layout_passes=False),
        scratch_types=[
            pltpu.VMEM((TOKENS,), jnp.int32),
            pltpu.VMEM((TOKENS,), jnp.int32),
            pltpu.VMEM((TOKENS,), jnp.int32),
            pltpu.VMEM((TOK_PER_W,), jnp.int32),
            pltpu.VMEM((LANES,), jnp.int32),
            pltpu.VMEM((BE_LEN,), jnp.int32),
            pltpu.VMEM((TOK_PER_W, HIDDEN), jnp.float32),
            pltpu.SemaphoreType.DMA,
        ],
    )(eidx, x_flat)


def _combine_body(y_hbm, pos_hbm, out_hbm, pos_v, rows_v, sem):
    wid = lax.axis_index("s") * NC + lax.axis_index("c")
    base = wid * TOK_PER_W
    pltpu.sync_copy(pos_hbm.at[pl.ds(base, TOK_PER_W)], pos_v)
    pltpu.async_copy(y_hbm.at[pos_v], rows_v, sem).wait()
    pltpu.sync_copy(rows_v, out_hbm.at[pl.ds(base, TOK_PER_W)])


def _combine(y_sorted, sorted_pos):
    return pl.kernel(
        _combine_body,
        out_type=jax.ShapeDtypeStruct((TOKENS, HIDDEN), jnp.float32),
        mesh=_SC_MESH,
        compiler_params=pltpu.CompilerParams(needs_layout_passes=False),
        scratch_types=[
            pltpu.VMEM((TOK_PER_W,), jnp.int32),
            pltpu.VMEM((TOK_PER_W, HIDDEN), jnp.float32),
            pltpu.SemaphoreType.DMA,
        ],
    )(y_sorted, sorted_pos)


def _mlp_body(be_ref, x_ref, w1_ref, b1_ref, w2_ref, b2_ref, o_ref):
    i = pl.program_id(0)
    nused = be_ref[NBLOCKS]

    @pl.when(i < nused)
    def _():
        h = lax.dot_general(
            x_ref[...], w1_ref[0], (((1,), (1,)), ((), ())),
            preferred_element_type=jnp.float32)
        h = jnp.maximum(h + b1_ref[0, 0], 0.0)
        o_ref[...] = lax.dot_general(
            h, w2_ref[0], (((1,), (1,)), ((), ())),
            preferred_element_type=jnp.float32) + b2_ref[0, 0]


def _expert_mlp(x_sorted, W1, b1, W2, b2, block_expert):
    grid_spec = pltpu.PrefetchScalarGridSpec(
        num_scalar_prefetch=1,
        grid=(NBLOCKS,),
        in_specs=[
            pl.BlockSpec((BLOCK, HIDDEN), lambda i, be: (i, 0)),
            pl.BlockSpec((1, EXPERT_SIZE, HIDDEN), lambda i, be: (be[i], 0, 0)),
            pl.BlockSpec((1, 1, EXPERT_SIZE), lambda i, be: (be[i], 0, 0)),
            pl.BlockSpec((1, HIDDEN, EXPERT_SIZE), lambda i, be: (be[i], 0, 0)),
            pl.BlockSpec((1, 1, HIDDEN), lambda i, be: (be[i], 0, 0)),
        ],
        out_specs=pl.BlockSpec((BLOCK, HIDDEN), lambda i, be: (i, 0)),
    )
    return pl.pallas_call(
        _mlp_body,
        grid_spec=grid_spec,
        out_shape=jax.ShapeDtypeStruct((PADDED, HIDDEN), jnp.float32),
        compiler_params=pltpu.CompilerParams(
            vmem_limit_bytes=110 * 1024 * 1024),
    )(block_expert, x_sorted, W1,
      b1.reshape(NUM_EXPERTS, 1, EXPERT_SIZE),
      W2, b2.reshape(NUM_EXPERTS, 1, HIDDEN))


def kernel(x, Wg, bg, W1, b1, W2, b2):
    batch, seq, hidden = x.shape
    x_flat = x.reshape(-1, hidden)
    eidx = _gate(x_flat, Wg, bg)
    sorted_pos, block_expert, x_sorted = _route_and_gather(eidx, x_flat)
    y_sorted = _expert_mlp(x_sorted, W1, b1, W2, b2, block_expert)
    out = _combine(y_sorted, sorted_pos)
    return out.reshape(batch, seq, hidden)
